# baseline (speedup 1.0000x reference)
import functools

import jax
import jax.numpy as jnp
from jax import lax
from jax.experimental import pallas as pl
from jax.experimental.pallas import tpu as pltpu
from jax.experimental.pallas import tpu_sc as plsc

G, N, E, D = 4, 10000, 160000, 128
H = D // 2
NC, NS = 2, 16
EPT = E // NS
K = 80
CHUNKS = 25
BLOCKS = EPT // (K * CHUNKS)
NBUF = 4
NPAD = 10240
RPT = NPAD // NS
ZR = 64


def _ln_msg_body(t_ref, gam_ref, bet_ref, x_ref, h_ref, s_ref):
    x = x_ref[...]
    mu = jnp.mean(x, axis=-1, keepdims=True)
    var = jnp.mean((x - mu) ** 2, axis=-1, keepdims=True)
    t = t_ref[0, 0]
    hh = (x - mu) / jnp.sqrt(var + 1e-5) * gam_ref[0] + bet_ref[0]
    h_ref[...] = hh
    p = jnp.maximum(hh, 0.0) + 1e-7
    q = jnp.exp(p * t)
    pq = p * q
    s_ref[0] = jnp.concatenate([pq[:, :H], q[:, :H]], axis=1)
    s_ref[1] = jnp.concatenate([pq[:, H:], q[:, H:]], axis=1)


def _ln_msg(x, gamma, beta, t):
    bn = 2000
    grid = (N // bn,)
    return pl.pallas_call(
        _ln_msg_body,
        grid=grid,
        in_specs=[
            pl.BlockSpec((1, 1), lambda nb: (0, 0)),
            pl.BlockSpec((1, D), lambda nb: (0, 0)),
            pl.BlockSpec((1, D), lambda nb: (0, 0)),
            pl.BlockSpec((bn, D), lambda nb: (nb, 0)),
        ],
        out_specs=[
            pl.BlockSpec((bn, D), lambda nb: (nb, 0)),
            pl.BlockSpec((2, bn, D), lambda nb: (0, nb, 0)),
        ],
        out_shape=[
            jax.ShapeDtypeStruct((N, D), jnp.float32),
            jax.ShapeDtypeStruct((2, N, D), jnp.float32),
        ],
    )(t.reshape(1, 1), gamma.reshape(1, D), beta.reshape(1, D), x)


def _sc_segsum(s3, src_r, dst_r):
    mesh = plsc.VectorSubcoreMesh(core_axis_name="c", subcore_axis_name="s")

    @functools.partial(
        pl.kernel,
        mesh=mesh,
        out_type=jax.ShapeDtypeStruct((2 * NPAD, D), jnp.float32),
        scratch_types=[
            pltpu.VMEM((CHUNKS, K), jnp.int32),
            pltpu.VMEM((CHUNKS, K), jnp.int32),
        ] + [pltpu.VMEM((K, D), jnp.float32) for _ in range(NBUF)] + [
            pltpu.VMEM_SHARED((NPAD, D), jnp.float32),
            pltpu.SemaphoreType.DMA,
            pltpu.SemaphoreType.DMA,
        ],
    )
    def k(s_hbm, src_hbm, dst_hbm, out_hbm, idx_s, idx_d, *rest):
        bufs = rest[:NBUF]
        rows0 = bufs[0]
        acc, sem_g, sem_s = rest[NBUF], rest[NBUF + 1], rest[NBUF + 2]
        c = lax.axis_index("c")
        s = lax.axis_index("s")

        def zrow(i, carry):
            for k16 in range(D // 16):
                rows0[i, pl.ds(k16 * 16, 16)] = jnp.zeros((16,), jnp.float32)
            return carry

        lax.fori_loop(0, ZR, zrow, 0)
        zcps = [pltpu.async_copy(
            rows0.at[pl.ds(0, ZR)],
            acc.at[pl.ds(s * RPT + w * ZR, ZR)], sem_s)
            for w in range(RPT // ZR)]
        for z in zcps:
            z.wait()
        plsc.subcore_barrier()

        s_g = s_hbm.at[c]

        def block_body(b, carry):
            pltpu.sync_copy(src_hbm.at[s, b], idx_s)
            pltpu.sync_copy(dst_hbm.at[s, b], idx_d)
            gath = [pltpu.async_copy(s_g.at[idx_s.at[j]], bufs[j], sem_g)
                    for j in range(min(NBUF - 1, CHUNKS))]
            scat = []
            for j in range(CHUNKS):
                gath[j].wait()
                if j >= 1:
                    scat[j - 1].wait()
                if j + NBUF - 1 < CHUNKS:
                    gath.append(pltpu.async_copy(
                        s_g.at[idx_s.at[j + NBUF - 1]],
                        bufs[(j + NBUF - 1) % NBUF], sem_g))
                scat.append(pltpu.async_copy(
                    bufs[j % NBUF], acc.at[idx_d.at[j]], sem_s, add=True))
            scat[CHUNKS - 1].wait()
            return carry

        lax.fori_loop(0, BLOCKS, block_body, 0)
        plsc.subcore_barrier()

        base = c * NPAD + s * RPT
        pltpu.sync_copy(acc.at[pl.ds(s * RPT, RPT)],
                        out_hbm.at[pl.ds(base, RPT)])

    return k(s3, src_r, dst_r)


def _combine_body(x_ref, h_ref, p_ref, w1_ref, b1_ref, g1_ref, be1_ref,
                  w2_ref, b2_ref, o_ref):
    p0 = p_ref[0]
    p1 = p_ref[1]
    agg_l = p0[:, :H] / (p0[:, H:] + 1e-16)
    agg_r = p1[:, :H] / (p1[:, H:] + 1e-16)
    on = jnp.concatenate([agg_l, agg_r], axis=1) + h_ref[...]
    z = jnp.dot(on, w1_ref[...], preferred_element_type=jnp.float32) + b1_ref[0]
    mu = jnp.mean(z, axis=-1, keepdims=True)
    var = jnp.mean((z - mu) ** 2, axis=-1, keepdims=True)
    z = (z - mu) / jnp.sqrt(var + 1e-5) * g1_ref[0] + be1_ref[0]
    z = jnp.maximum(z, 0.0)
    y = jnp.dot(z, w2_ref[...], preferred_element_type=jnp.float32) + b2_ref[0]
    o_ref[...] = x_ref[...] + y


def _combine_mlp(x, h, partial, w1, b1, g1, be1, w2, b2):
    bm = 2000
    grid = (N // bm,)
    d2 = 2 * D
    return pl.pallas_call(
        _combine_body,
        grid=grid,
        in_specs=[
            pl.BlockSpec((bm, D), lambda nb: (nb, 0)),
            pl.BlockSpec((bm, D), lambda nb: (nb, 0)),
            pl.BlockSpec((2, bm, D), lambda nb: (0, nb, 0)),
            pl.BlockSpec((D, d2), lambda nb: (0, 0)),
            pl.BlockSpec((1, d2), lambda nb: (0, 0)),
            pl.BlockSpec((1, d2), lambda nb: (0, 0)),
            pl.BlockSpec((1, d2), lambda nb: (0, 0)),
            pl.BlockSpec((d2, D), lambda nb: (0, 0)),
            pl.BlockSpec((1, D), lambda nb: (0, 0)),
        ],
        out_specs=pl.BlockSpec((bm, D), lambda nb: (nb, 0)),
        out_shape=jax.ShapeDtypeStruct((N, D), jnp.float32),
    )(x, h, partial, w1, b1.reshape(1, d2), g1.reshape(1, d2),
      be1.reshape(1, d2), w2, b2.reshape(1, D))


def kernel(input_nodes, input_edges, ln_gamma, ln_beta, t, W1, b1,
           mlp_ln_gamma, mlp_ln_beta, W2, b2):
    edges = input_edges.astype(jnp.int32)
    src_r = edges[:, 0, :].reshape(G, NS, BLOCKS, CHUNKS, K)
    dst_r = edges[:, 1, :].reshape(G, NS, BLOCKS, CHUNKS, K)
    tf = t.astype(jnp.float32)

    hs = [_ln_msg(input_nodes[g], ln_gamma, ln_beta, tf) for g in range(G)]
    parts = [_sc_segsum(hs[g][1], src_r[g], dst_r[g]) for g in range(G)]
    outs = [_combine_mlp(
        input_nodes[g], hs[g][0], parts[g].reshape(2, NPAD, D),
        W1, b1, mlp_ln_gamma, mlp_ln_beta, W2, b2) for g in range(G)]
    return jnp.stack(outs)


# speedup vs baseline: 1.0024x; 1.0024x over previous
import functools

import jax
import jax.numpy as jnp
from jax import lax
from jax.experimental import pallas as pl
from jax.experimental.pallas import tpu as pltpu
from jax.experimental.pallas import tpu_sc as plsc

G, N, E, D = 4, 10000, 160000, 128
H = D // 2
NC, NS = 2, 16
EPT = E // NS
K = 80
CHUNKS = 25
BLOCKS = EPT // (K * CHUNKS)
NBUF = 4
NPAD = 10240
RPT = NPAD // NS
ZR = 64


def _ln_msg_body(t_ref, gam_ref, bet_ref, x_ref, h_ref, s_ref):
    x = x_ref[...]
    mu = jnp.mean(x, axis=-1, keepdims=True)
    var = jnp.mean((x - mu) ** 2, axis=-1, keepdims=True)
    t = t_ref[0, 0]
    hh = (x - mu) / jnp.sqrt(var + 1e-5) * gam_ref[0] + bet_ref[0]
    h_ref[...] = hh
    p = jnp.maximum(hh, 0.0) + 1e-7
    q = jnp.exp(p * t)
    pq = p * q
    s_ref[0] = jnp.concatenate([pq[:, :H], q[:, :H]], axis=1)
    s_ref[1] = jnp.concatenate([pq[:, H:], q[:, H:]], axis=1)


def _ln_msg(x, gamma, beta, t):
    bn = 1000
    grid = (N // bn,)
    return pl.pallas_call(
        _ln_msg_body,
        grid=grid,
        in_specs=[
            pl.BlockSpec((1, 1), lambda nb: (0, 0)),
            pl.BlockSpec((1, D), lambda nb: (0, 0)),
            pl.BlockSpec((1, D), lambda nb: (0, 0)),
            pl.BlockSpec((bn, D), lambda nb: (nb, 0)),
        ],
        out_specs=[
            pl.BlockSpec((bn, D), lambda nb: (nb, 0)),
            pl.BlockSpec((2, bn, D), lambda nb: (0, nb, 0)),
        ],
        out_shape=[
            jax.ShapeDtypeStruct((N, D), jnp.float32),
            jax.ShapeDtypeStruct((2, N, D), jnp.float32),
        ],
    )(t.reshape(1, 1), gamma.reshape(1, D), beta.reshape(1, D), x)


def _sc_segsum(s3, src_r, dst_r):
    mesh = plsc.VectorSubcoreMesh(core_axis_name="c", subcore_axis_name="s")

    @functools.partial(
        pl.kernel,
        mesh=mesh,
        out_type=jax.ShapeDtypeStruct((2 * NPAD, D), jnp.float32),
        scratch_types=[
            pltpu.VMEM((CHUNKS, K), jnp.int32),
            pltpu.VMEM((CHUNKS, K), jnp.int32),
        ] + [pltpu.VMEM((K, D), jnp.float32) for _ in range(NBUF)] + [
            pltpu.VMEM_SHARED((NPAD, D), jnp.float32),
            pltpu.SemaphoreType.DMA,
            pltpu.SemaphoreType.DMA,
        ],
    )
    def k(s_hbm, src_hbm, dst_hbm, out_hbm, idx_s, idx_d, *rest):
        bufs = rest[:NBUF]
        rows0 = bufs[0]
        acc, sem_g, sem_s = rest[NBUF], rest[NBUF + 1], rest[NBUF + 2]
        c = lax.axis_index("c")
        s = lax.axis_index("s")

        def zrow(i, carry):
            for k16 in range(D // 16):
                rows0[i, pl.ds(k16 * 16, 16)] = jnp.zeros((16,), jnp.float32)
            return carry

        lax.fori_loop(0, ZR, zrow, 0)
        zcps = [pltpu.async_copy(
            rows0.at[pl.ds(0, ZR)],
            acc.at[pl.ds(s * RPT + w * ZR, ZR)], sem_s)
            for w in range(RPT // ZR)]
        for z in zcps:
            z.wait()
        plsc.subcore_barrier()

        s_g = s_hbm.at[c]

        def block_body(b, carry):
            pltpu.sync_copy(src_hbm.at[s, b], idx_s)
            pltpu.sync_copy(dst_hbm.at[s, b], idx_d)
            gath = [pltpu.async_copy(s_g.at[idx_s.at[j]], bufs[j], sem_g)
                    for j in range(min(NBUF - 1, CHUNKS))]
            scat = []
            for j in range(CHUNKS):
                gath[j].wait()
                if j >= 1:
                    scat[j - 1].wait()
                if j + NBUF - 1 < CHUNKS:
                    gath.append(pltpu.async_copy(
                        s_g.at[idx_s.at[j + NBUF - 1]],
                        bufs[(j + NBUF - 1) % NBUF], sem_g))
                scat.append(pltpu.async_copy(
                    bufs[j % NBUF], acc.at[idx_d.at[j]], sem_s, add=True))
            scat[CHUNKS - 1].wait()
            return carry

        lax.fori_loop(0, BLOCKS, block_body, 0)
        plsc.subcore_barrier()

        base = c * NPAD + s * RPT
        pltpu.sync_copy(acc.at[pl.ds(s * RPT, RPT)],
                        out_hbm.at[pl.ds(base, RPT)])

    return k(s3, src_r, dst_r)


def _combine_body(x_ref, h_ref, p_ref, w1_ref, b1_ref, g1_ref, be1_ref,
                  w2_ref, b2_ref, o_ref):
    p0 = p_ref[0]
    p1 = p_ref[1]
    agg_l = p0[:, :H] / (p0[:, H:] + 1e-16)
    agg_r = p1[:, :H] / (p1[:, H:] + 1e-16)
    on = jnp.concatenate([agg_l, agg_r], axis=1) + h_ref[...]
    z = jnp.dot(on, w1_ref[...], preferred_element_type=jnp.float32) + b1_ref[0]
    mu = jnp.mean(z, axis=-1, keepdims=True)
    var = jnp.mean((z - mu) ** 2, axis=-1, keepdims=True)
    z = (z - mu) / jnp.sqrt(var + 1e-5) * g1_ref[0] + be1_ref[0]
    z = jnp.maximum(z, 0.0)
    y = jnp.dot(z, w2_ref[...], preferred_element_type=jnp.float32) + b2_ref[0]
    o_ref[...] = x_ref[...] + y


def _combine_mlp(x, h, partial, w1, b1, g1, be1, w2, b2):
    bm = 1000
    grid = (N // bm,)
    d2 = 2 * D
    return pl.pallas_call(
        _combine_body,
        grid=grid,
        in_specs=[
            pl.BlockSpec((bm, D), lambda nb: (nb, 0)),
            pl.BlockSpec((bm, D), lambda nb: (nb, 0)),
            pl.BlockSpec((2, bm, D), lambda nb: (0, nb, 0)),
            pl.BlockSpec((D, d2), lambda nb: (0, 0)),
            pl.BlockSpec((1, d2), lambda nb: (0, 0)),
            pl.BlockSpec((1, d2), lambda nb: (0, 0)),
            pl.BlockSpec((1, d2), lambda nb: (0, 0)),
            pl.BlockSpec((d2, D), lambda nb: (0, 0)),
            pl.BlockSpec((1, D), lambda nb: (0, 0)),
        ],
        out_specs=pl.BlockSpec((bm, D), lambda nb: (nb, 0)),
        out_shape=jax.ShapeDtypeStruct((N, D), jnp.float32),
    )(x, h, partial, w1, b1.reshape(1, d2), g1.reshape(1, d2),
      be1.reshape(1, d2), w2, b2.reshape(1, D))


def kernel(input_nodes, input_edges, ln_gamma, ln_beta, t, W1, b1,
           mlp_ln_gamma, mlp_ln_beta, W2, b2):
    edges = input_edges.astype(jnp.int32)
    src_r = edges[:, 0, :].reshape(G, NS, BLOCKS, CHUNKS, K)
    dst_r = edges[:, 1, :].reshape(G, NS, BLOCKS, CHUNKS, K)
    tf = t.astype(jnp.float32)

    hs = [_ln_msg(input_nodes[g], ln_gamma, ln_beta, tf) for g in range(G)]
    parts = [_sc_segsum(hs[g][1], src_r[g], dst_r[g]) for g in range(G)]
    outs = [_combine_mlp(
        input_nodes[g], hs[g][0], parts[g].reshape(2, NPAD, D),
        W1, b1, mlp_ln_gamma, mlp_ln_beta, W2, b2) for g in range(G)]
    return jnp.stack(outs)
